# Initial kernel scaffold; baseline (speedup 1.0000x reference)
import jax
import jax.numpy as jnp
from jax.experimental import pallas as pl


def kernel(x, params, edge_index, batch, keypoints):
    raise NotImplementedError("write your pallas kernel here")


# speedup vs baseline: 31.4424x; 31.4424x over previous
import functools
import jax
import jax.numpy as jnp
from jax import lax
from jax.experimental import pallas as pl
from jax.experimental.pallas import tpu as pltpu
from jax.experimental.pallas import tpu_sc as plsc

N = 10000
E = 320000
H = 128
K = 4
DH = 32
G = 128
NC = 10
BN = 2000
GRID = N // BN


def _rows(i):
    return (i, 0)


def _full(i):
    return (0, 0)


def _k_stage0(x_ref, g_ref, b_ref, W_ref, c_ref, o_ref):
    x = x_ref[...]
    m = jnp.mean(x, axis=-1, keepdims=True)
    v = jnp.mean((x - m) ** 2, axis=-1, keepdims=True)
    ln = (x - m) * jax.lax.rsqrt(v + 1e-5) * g_ref[...] + b_ref[...]
    h = jnp.dot(ln, W_ref[...], preferred_element_type=jnp.float32) + c_ref[...]
    o_ref[...] = jnp.maximum(h, 0.0)


def _stage0(x, g, b, W, c):
    return pl.pallas_call(
        _k_stage0,
        grid=(GRID,),
        in_specs=[
            pl.BlockSpec((BN, H), _rows),
            pl.BlockSpec((1, H), _full),
            pl.BlockSpec((1, H), _full),
            pl.BlockSpec((H, H), _full),
            pl.BlockSpec((1, H), _full),
        ],
        out_specs=pl.BlockSpec((BN, H), _rows),
        out_shape=jax.ShapeDtypeStruct((N, H), jnp.float32),
    )(x, g.reshape(1, H), b.reshape(1, H), W, c.reshape(1, H))


def _k_gat_pre(h_ref, W_ref, c_ref, A_ref, hw_ref, aa_ref):
    hw = jnp.dot(h_ref[...], W_ref[...], preferred_element_type=jnp.float32) + c_ref[...]
    hw_ref[...] = hw
    aa_ref[...] = jnp.dot(hw, A_ref[...], preferred_element_type=jnp.float32)


def _gat_pre(h, Wp, cp, Aall):
    return pl.pallas_call(
        _k_gat_pre,
        grid=(GRID,),
        in_specs=[
            pl.BlockSpec((BN, H), _rows),
            pl.BlockSpec((H, H), _full),
            pl.BlockSpec((1, H), _full),
            pl.BlockSpec((H, 8), _full),
        ],
        out_specs=[
            pl.BlockSpec((BN, H), _rows),
            pl.BlockSpec((BN, 8), _rows),
        ],
        out_shape=[
            jax.ShapeDtypeStruct((N, H), jnp.float32),
            jax.ShapeDtypeStruct((N, 8), jnp.float32),
        ],
    )(h, Wp, cp.reshape(1, H), Aall)


def _k_gat_post(out2_ref, den2_ref, aa_ref, hw_ref, b_ref, o_ref):
    outs = out2_ref[0] + out2_ref[1]
    dens = den2_ref[0] + den2_ref[1]
    aa = aa_ref[...]
    tl = aa[:, 0:4] + aa[:, 4:8]
    al = jnp.maximum(tl, 0.0) + 0.2 * jnp.minimum(tl, 0.0)
    exl = jnp.exp(al)
    den = dens + exl
    hw = hw_ref[...]
    exb = jnp.repeat(exl, DH, axis=1)
    denb = jnp.repeat(den, DH, axis=1)
    o = (outs + exb * hw) / denb + b_ref[...]
    o_ref[...] = jnp.maximum(o, 0.0)


def _gat_post(out2, den2, aa, hw, bias):
    return pl.pallas_call(
        _k_gat_post,
        grid=(GRID,),
        in_specs=[
            pl.BlockSpec((2, BN, H), lambda i: (0, i, 0)),
            pl.BlockSpec((2, BN, K), lambda i: (0, i, 0)),
            pl.BlockSpec((BN, 8), _rows),
            pl.BlockSpec((BN, H), _rows),
            pl.BlockSpec((1, H), _full),
        ],
        out_specs=pl.BlockSpec((BN, H), _rows),
        out_shape=jax.ShapeDtypeStruct((N, H), jnp.float32),
    )(out2, den2, aa, hw, bias.reshape(1, H))


def _k_final(h_ref, We_ref, eb_ref, Wn_ref, nb_ref, Wc_ref, cc_ref,
             Wo_ref, co_ref, suv_ref, hc_ref, ho_ref):
    h = h_ref[...]
    uv = jnp.dot(h, We_ref[...], preferred_element_type=jnp.float32) + eb_ref[...]
    su = uv[:, 0:1] - uv[:, 1:2]
    sv = uv[:, 2:3] - uv[:, 3:4]
    suv = jnp.concatenate([su, sv, jnp.zeros_like(uv[:, 0:6])], axis=1)
    suv_ref[...] = suv
    nl = jnp.dot(h, Wn_ref[...], preferred_element_type=jnp.float32) + nb_ref[...]
    m = jnp.maximum(nl[:, 0:1], nl[:, 1:2])
    e0 = jnp.exp(nl[:, 0:1] - m)
    e1 = jnp.exp(nl[:, 1:2] - m)
    na0 = e0 / (e0 + e1)
    na1 = 1.0 - na0
    xc = na0 * h
    xo = na1 * h
    hc_ref[...] = jnp.dot(xc, Wc_ref[...], preferred_element_type=jnp.float32) + cc_ref[...]
    ho_ref[...] = jnp.dot(xo, Wo_ref[...], preferred_element_type=jnp.float32) + co_ref[...]


def _final(h, We, eb, Wn, nb, Wc, cc, Wo, co):
    return pl.pallas_call(
        _k_final,
        grid=(GRID,),
        in_specs=[
            pl.BlockSpec((BN, H), _rows),
            pl.BlockSpec((H, 8), _full),
            pl.BlockSpec((1, 8), _full),
            pl.BlockSpec((H, 8), _full),
            pl.BlockSpec((1, 8), _full),
            pl.BlockSpec((H, H), _full),
            pl.BlockSpec((1, H), _full),
            pl.BlockSpec((H, H), _full),
            pl.BlockSpec((1, H), _full),
        ],
        out_specs=[
            pl.BlockSpec((BN, 8), _rows),
            pl.BlockSpec((BN, H), _rows),
            pl.BlockSpec((BN, H), _rows),
        ],
        out_shape=[
            jax.ShapeDtypeStruct((N, 8), jnp.float32),
            jax.ShapeDtypeStruct((N, H), jnp.float32),
            jax.ShapeDtypeStruct((N, H), jnp.float32),
        ],
    )(h, We, eb, Wn, nb, Wc, cc.reshape(1, H), Wo, co.reshape(1, H))


def _k_dinv(deg2_ref, o_ref):
    d = 1.0 + deg2_ref[0] + deg2_ref[1]
    o_ref[...] = jax.lax.rsqrt(d)


def _dinv(deg2):
    return pl.pallas_call(
        _k_dinv,
        grid=(GRID,),
        in_specs=[pl.BlockSpec((2, BN, 2), lambda i: (0, i, 0))],
        out_specs=pl.BlockSpec((BN, 2), _rows),
        out_shape=jax.ShapeDtypeStruct((N, 2), jnp.float32),
    )(deg2)


def _k_pool(outc2_ref, outo2_ref, hc_ref, ho_ref, dinv_ref, batch_ref,
            cb_ref, ob_ref, hp_ref,
            lc_ref, lo_ref, po_ref, pc_acc):
    i = pl.program_id(0)

    @pl.when(i == 0)
    def _init():
        pc_acc[...] = jnp.zeros_like(pc_acc)
        po_ref[...] = jnp.zeros_like(po_ref)

    dinv = dinv_ref[...]
    d2c = dinv[:, 0:1] * dinv[:, 0:1]
    d2o = dinv[:, 1:2] * dinv[:, 1:2]
    xcn = jnp.maximum(outc2_ref[0] + outc2_ref[1] + d2c * hc_ref[...] + cb_ref[...], 0.0)
    xon = jnp.maximum(outo2_ref[0] + outo2_ref[1] + d2o * ho_ref[...] + ob_ref[...], 0.0)
    gids = jax.lax.broadcasted_iota(jnp.int32, (G, BN), 0)
    onehot = (gids == batch_ref[0]).astype(jnp.float32)
    pc_acc[...] += jnp.dot(onehot, xcn, preferred_element_type=jnp.float32)
    po_ref[...] += jnp.dot(onehot, xon, preferred_element_type=jnp.float32)

    @pl.when(i == GRID - 1)
    def _heads():
        hp = hp_ref[...]

        def ln(z, g, b):
            m = jnp.mean(z, axis=-1, keepdims=True)
            v = jnp.mean((z - m) ** 2, axis=-1, keepdims=True)
            return (z - m) * jax.lax.rsqrt(v + 1e-5) * g + b

        def head(pool, base):
            z = ln(pool, hp[base + 0:base + 1], hp[base + 1:base + 2])
            z = jnp.maximum(jnp.dot(z, hp[base + 2:base + 130],
                                    preferred_element_type=jnp.float32)
                            + hp[base + 130:base + 131], 0.0)
            z = ln(z, hp[base + 131:base + 132], hp[base + 132:base + 133])
            lg = jnp.dot(z, hp[base + 133:base + 261],
                         preferred_element_type=jnp.float32)[:, :NC] \
                + hp[base + 261:base + 262, :NC]
            return lg

        lc_ref[...] = head(pc_acc[...], 0)
        lo_ref[...] = head(po_ref[...], 262)


def _pool_heads(outc2, outo2, hc, ho, dinv, batch2d, cb, ob, headpack):
    return pl.pallas_call(
        _k_pool,
        grid=(GRID,),
        in_specs=[
            pl.BlockSpec((2, BN, H), lambda i: (0, i, 0)),
            pl.BlockSpec((2, BN, H), lambda i: (0, i, 0)),
            pl.BlockSpec((BN, H), _rows),
            pl.BlockSpec((BN, H), _rows),
            pl.BlockSpec((BN, 2), _rows),
            pl.BlockSpec((1, 1, BN), lambda i: (i, 0, 0)),
            pl.BlockSpec((1, H), _full),
            pl.BlockSpec((1, H), _full),
            pl.BlockSpec((524, H), _full),
        ],
        out_specs=[
            pl.BlockSpec((G, NC), _full),
            pl.BlockSpec((G, NC), _full),
            pl.BlockSpec((G, H), _full),
        ],
        out_shape=[
            jax.ShapeDtypeStruct((G, NC), jnp.float32),
            jax.ShapeDtypeStruct((G, NC), jnp.float32),
            jax.ShapeDtypeStruct((G, H), jnp.float32),
        ],
        scratch_shapes=[pltpu.VMEM((G, H), jnp.float32)],
    )(outc2, outo2, hc, ho, dinv, batch2d, cb.reshape(1, H), ob.reshape(1, H),
      headpack)


SC_C = 2
SC_T = 16
NW = SC_C * SC_T
EW = E // NW
CH = 80
NCHUNK = EW // CH
NPAD = 10240
ZROWS = NPAD // SC_T
ZSTEP = ZROWS // CH

_PE = tuple([e for e in range(4) for _ in range(4)])
_HM = tuple(list(range(4)) * 4)


def _sc_gat_body(hw_hbm, aa_hbm, src_hbm, dst_hbm, out_hbm, den_hbm,
                 src_v, dst_v, si_v, ddi_v, di_v, ex_v, asg_v, adg_v,
                 rows_v, out_sh, den_sh, sem):
    c = lax.axis_index("c")
    s = lax.axis_index("s")
    w = s * SC_C + c
    iota = lax.iota(jnp.int32, 16)
    pe = iota >> 2
    hm = iota & 3
    zero = jnp.full((16,), 0.0, jnp.float32)

    def _zr(j, _):
        for t in range(H // 16):
            rows_v[j, pl.ds(16 * t, 16)] = zero
        return 0
    lax.fori_loop(0, CH, _zr, 0)

    def _ze(q, _):
        ex_v[pl.ds(16 * q, 16)] = zero
        return 0
    lax.fori_loop(0, CH * K // 16, _ze, 0)

    for i in range(ZSTEP):
        start = s * ZROWS + i * CH
        pltpu.sync_copy(rows_v, out_sh.at[pl.ds(start, CH)])
        pltpu.sync_copy(ex_v, den_sh.at[pl.ds(K * start, K * CH)])
    plsc.subcore_barrier()

    def _chunk(g, _):
        base = w * EW + g * CH
        pltpu.sync_copy(src_hbm.at[pl.ds(base, CH)], src_v)
        pltpu.sync_copy(dst_hbm.at[pl.ds(base, CH)], dst_v)
        hrows = pltpu.async_copy(hw_hbm.at[src_v], rows_v, sem)

        def _idx4(q, _):
            idx_e = pe + 4 * q
            srcs = plsc.load_gather(src_v, [idx_e])
            dsts = plsc.load_gather(dst_v, [idx_e])
            si_v[pl.ds(16 * q, 16)] = srcs * 8 + hm
            ddi_v[pl.ds(16 * q, 16)] = dsts * 8 + (hm + 4)
            di_v[pl.ds(16 * q, 16)] = dsts * K + hm
            return 0
        lax.fori_loop(0, CH // 4, _idx4, 0)

        pltpu.sync_copy(aa_hbm.at[si_v], asg_v)
        pltpu.sync_copy(aa_hbm.at[ddi_v], adg_v)

        def _ex16(q, _):
            t = asg_v[pl.ds(16 * q, 16)] + adg_v[pl.ds(16 * q, 16)]
            alpha = jnp.maximum(t, 0.0) + 0.2 * jnp.minimum(t, 0.0)
            ex_v[pl.ds(16 * q, 16)] = jnp.exp(alpha)
            return 0
        lax.fori_loop(0, CH * K // 16, _ex16, 0)

        hrows.wait()

        def _scale4(q, _):
            exv = ex_v[pl.ds(16 * q, 16)]
            for j in range(4):
                e = 4 * q + j
                for k in range(K):
                    sel = jnp.full((16,), 4 * j + k, jnp.int32)
                    sc = exv.at[sel].get(mode="promise_in_bounds")
                    for t2 in range(2):
                        col = k * DH + 16 * t2
                        rows_v[e, pl.ds(col, 16)] = rows_v[e, pl.ds(col, 16)] * sc
            return 0
        lax.fori_loop(0, CH // 4, _scale4, 0)

        pltpu.sync_copy(ex_v, den_sh.at[di_v], add=True)
        pltpu.sync_copy(rows_v, out_sh.at[dst_v], add=True)
        return 0

    lax.fori_loop(0, NCHUNK, _chunk, 0)
    plsc.subcore_barrier()

    for i in range(ZSTEP):
        start = s * ZROWS + i * CH

        @pl.when(start < N)
        def _copy_out():
            pltpu.sync_copy(out_sh.at[pl.ds(start, CH)], rows_v)
            pltpu.sync_copy(rows_v, out_hbm.at[c, pl.ds(start, CH)])
            pltpu.sync_copy(den_sh.at[pl.ds(K * start, K * CH)], ex_v)
            pltpu.sync_copy(ex_v,
                            den_hbm.at[pl.ds(c * (N * K) + K * start, K * CH)])


def _edge_gat_sc(hw, aa, src, dst):
    mesh = plsc.VectorSubcoreMesh(core_axis_name="c", subcore_axis_name="s")
    f = pl.kernel(
        _sc_gat_body,
        mesh=mesh,
        compiler_params=pltpu.CompilerParams(needs_layout_passes=False),
        out_type=[
            jax.ShapeDtypeStruct((SC_C, N, H), jnp.float32),
            jax.ShapeDtypeStruct((SC_C * N * K,), jnp.float32),
        ],
        scratch_types=[
            pltpu.VMEM((CH,), jnp.int32),
            pltpu.VMEM((CH,), jnp.int32),
            pltpu.VMEM((CH * K,), jnp.int32),
            pltpu.VMEM((CH * K,), jnp.int32),
            pltpu.VMEM((CH * K,), jnp.int32),
            pltpu.VMEM((CH * K,), jnp.float32),
            pltpu.VMEM((CH * K,), jnp.float32),
            pltpu.VMEM((CH * K,), jnp.float32),
            pltpu.VMEM((CH, H), jnp.float32),
            pltpu.VMEM_SHARED((NPAD, H), jnp.float32),
            pltpu.VMEM_SHARED((NPAD * K,), jnp.float32),
            pltpu.SemaphoreType.DMA,
        ],
    )
    out2, denf = f(hw, aa.reshape(N * 8), src, dst)
    return out2, denf.reshape(SC_C, N, K)


def _sc_att_body(suv_hbm, src_hbm, dst_hbm, ew_hbm, deg_hbm,
                 src_v, dst_v, si_v, dvi_v, di_v, sug_v, svg_v, ew_v,
                 deg_sh, sem):
    c = lax.axis_index("c")
    s = lax.axis_index("s")
    w = s * SC_C + c
    iota = lax.iota(jnp.int32, 16)
    pe2 = iota >> 1
    m2 = iota & 1
    zero = jnp.full((16,), 0.0, jnp.float32)

    def _zw(q, _):
        ew_v[pl.ds(16 * q, 16)] = zero
        return 0
    lax.fori_loop(0, CH * 2 // 16, _zw, 0)
    for i in range(ZSTEP):
        start = s * ZROWS + i * CH
        pltpu.sync_copy(ew_v, deg_sh.at[pl.ds(2 * start, 2 * CH)])
    plsc.subcore_barrier()

    def _chunk(g, _):
        base = w * EW + g * CH
        pltpu.sync_copy(src_hbm.at[pl.ds(base, CH)], src_v)
        pltpu.sync_copy(dst_hbm.at[pl.ds(base, CH)], dst_v)

        def _idx8(q, _):
            idx_e = pe2 + 8 * q
            srcs = plsc.load_gather(src_v, [idx_e])
            dsts = plsc.load_gather(dst_v, [idx_e])
            si_v[pl.ds(16 * q, 16)] = srcs * 8
            dvi_v[pl.ds(16 * q, 16)] = dsts * 8 + 1
            di_v[pl.ds(16 * q, 16)] = srcs * 2 + m2
            return 0
        lax.fori_loop(0, CH * 2 // 16, _idx8, 0)

        pltpu.sync_copy(suv_hbm.at[si_v], sug_v)
        pltpu.sync_copy(suv_hbm.at[dvi_v], svg_v)

        def _cw(q, _):
            t = sug_v[pl.ds(16 * q, 16)] + svg_v[pl.ds(16 * q, 16)]
            ewc = 1.0 / (1.0 + jnp.exp(-t))
            ew_v[pl.ds(16 * q, 16)] = jnp.where(m2 == 0, ewc, 1.0 - ewc)
            return 0
        lax.fori_loop(0, CH * 2 // 16, _cw, 0)

        pltpu.sync_copy(ew_v, ew_hbm.at[pl.ds(2 * base, 2 * CH)])
        pltpu.sync_copy(ew_v, deg_sh.at[di_v], add=True)
        return 0

    lax.fori_loop(0, NCHUNK, _chunk, 0)
    plsc.subcore_barrier()
    for i in range(ZSTEP):
        start = s * ZROWS + i * CH

        @pl.when(start < N)
        def _copy_out():
            pltpu.sync_copy(deg_sh.at[pl.ds(2 * start, 2 * CH)], ew_v)
            pltpu.sync_copy(ew_v,
                            deg_hbm.at[pl.ds(c * (N * 2) + 2 * start, 2 * CH)])


def _edge_att_deg_sc(suv, src, dst):
    mesh = plsc.VectorSubcoreMesh(core_axis_name="c", subcore_axis_name="s")
    f = pl.kernel(
        _sc_att_body,
        mesh=mesh,
        compiler_params=pltpu.CompilerParams(needs_layout_passes=False),
        out_type=[
            jax.ShapeDtypeStruct((E * 2,), jnp.float32),
            jax.ShapeDtypeStruct((SC_C * N * 2,), jnp.float32),
        ],
        scratch_types=[
            pltpu.VMEM((CH,), jnp.int32),
            pltpu.VMEM((CH,), jnp.int32),
            pltpu.VMEM((CH * 2,), jnp.int32),
            pltpu.VMEM((CH * 2,), jnp.int32),
            pltpu.VMEM((CH * 2,), jnp.int32),
            pltpu.VMEM((CH * 2,), jnp.float32),
            pltpu.VMEM((CH * 2,), jnp.float32),
            pltpu.VMEM((CH * 2,), jnp.float32),
            pltpu.VMEM_SHARED((NPAD * 2,), jnp.float32),
            pltpu.SemaphoreType.DMA,
        ],
    )
    ew, degf = f(suv.reshape(N * 8), src, dst)
    return ew, degf.reshape(SC_C, N, 2)


def _make_gcn_body(off):
    def body(h_hbm, dinv_hbm, ew_hbm, src_hbm, dst_hbm, out_hbm,
             src_v, dst_v, si_v, dvi_v, ewi_v, ds_v, dd_v, ewv_v, nm_v,
             rows_v, out_sh, sem):
        c = lax.axis_index("c")
        s = lax.axis_index("s")
        w = s * SC_C + c
        iota = lax.iota(jnp.int32, 16)
        zero = jnp.full((16,), 0.0, jnp.float32)

        def _zr(j, _):
            for t in range(H // 16):
                rows_v[j, pl.ds(16 * t, 16)] = zero
            return 0
        lax.fori_loop(0, CH, _zr, 0)
        for i in range(ZSTEP):
            start = s * ZROWS + i * CH
            pltpu.sync_copy(rows_v, out_sh.at[pl.ds(start, CH)])
        plsc.subcore_barrier()

        def _chunk(g, _):
            base = w * EW + g * CH
            pltpu.sync_copy(src_hbm.at[pl.ds(base, CH)], src_v)
            pltpu.sync_copy(dst_hbm.at[pl.ds(base, CH)], dst_v)
            hrows = pltpu.async_copy(h_hbm.at[src_v], rows_v, sem)

            def _idx16(q, _):
                idx_e = iota + 16 * q
                srcs = plsc.load_gather(src_v, [idx_e])
                dsts = plsc.load_gather(dst_v, [idx_e])
                si_v[pl.ds(16 * q, 16)] = srcs * 2 + off
                dvi_v[pl.ds(16 * q, 16)] = dsts * 2 + off
                ewi_v[pl.ds(16 * q, 16)] = (base + idx_e) * 2 + off
                return 0
            lax.fori_loop(0, CH // 16, _idx16, 0)

            pltpu.sync_copy(dinv_hbm.at[si_v], ds_v)
            pltpu.sync_copy(dinv_hbm.at[dvi_v], dd_v)
            pltpu.sync_copy(ew_hbm.at[ewi_v], ewv_v)

            def _nm(q, _):
                sl = pl.ds(16 * q, 16)
                nm_v[sl] = ds_v[sl] * ewv_v[sl] * dd_v[sl]
                return 0
            lax.fori_loop(0, CH // 16, _nm, 0)

            hrows.wait()

            def _scale(q, _):
                nv = nm_v[pl.ds(16 * q, 16)]
                for j in range(16):
                    e = 16 * q + j
                    sel = jnp.full((16,), j, jnp.int32)
                    sc = nv.at[sel].get(mode="promise_in_bounds")
                    for t2 in range(H // 16):
                        col = 16 * t2
                        rows_v[e, pl.ds(col, 16)] = rows_v[e, pl.ds(col, 16)] * sc
                return 0
            lax.fori_loop(0, CH // 16, _scale, 0)

            pltpu.sync_copy(rows_v, out_sh.at[dst_v], add=True)
            return 0

        lax.fori_loop(0, NCHUNK, _chunk, 0)
        plsc.subcore_barrier()
        for i in range(ZSTEP):
            start = s * ZROWS + i * CH

            @pl.when(start < N)
            def _copy_out():
                pltpu.sync_copy(out_sh.at[pl.ds(start, CH)], rows_v)
                pltpu.sync_copy(rows_v, out_hbm.at[c, pl.ds(start, CH)])
    return body


def _edge_gcn_sc(h, dinv, ew, src, dst, off):
    mesh = plsc.VectorSubcoreMesh(core_axis_name="c", subcore_axis_name="s")
    f = pl.kernel(
        _make_gcn_body(off),
        mesh=mesh,
        compiler_params=pltpu.CompilerParams(needs_layout_passes=False),
        out_type=jax.ShapeDtypeStruct((SC_C, N, H), jnp.float32),
        scratch_types=[
            pltpu.VMEM((CH,), jnp.int32),
            pltpu.VMEM((CH,), jnp.int32),
            pltpu.VMEM((CH,), jnp.int32),
            pltpu.VMEM((CH,), jnp.int32),
            pltpu.VMEM((CH,), jnp.int32),
            pltpu.VMEM((CH,), jnp.float32),
            pltpu.VMEM((CH,), jnp.float32),
            pltpu.VMEM((CH,), jnp.float32),
            pltpu.VMEM((CH,), jnp.float32),
            pltpu.VMEM((CH, H), jnp.float32),
            pltpu.VMEM_SHARED((NPAD, H), jnp.float32),
            pltpu.SemaphoreType.DMA,
        ],
    )
    return f(h, dinv.reshape(N * 2), ew, src, dst)


def _edge_gat(hw, aa, src, dst):
    t = aa[src, 0:4] + aa[dst, 4:8]
    alpha = jnp.maximum(t, 0) + 0.2 * jnp.minimum(t, 0)
    ex = jnp.exp(alpha)
    den = jnp.zeros((N, K), jnp.float32).at[dst].add(ex)
    hw4 = hw.reshape(N, K, DH)
    out = jnp.zeros((N, K, DH), jnp.float32).at[dst].add(ex[:, :, None] * hw4[src])
    out2 = jnp.stack([out.reshape(N, H), jnp.zeros((N, H), jnp.float32)])
    den2 = jnp.stack([den, jnp.zeros((N, K), jnp.float32)])
    return out2, den2


def _edge_att_deg(suv, src, dst):
    t = suv[src, 0] + suv[dst, 1]
    ewc = jax.nn.sigmoid(t)
    ewo = 1.0 - ewc
    ew = jnp.stack([ewc, ewo], axis=1)
    deg = jnp.zeros((N, 2), jnp.float32).at[src].add(ew)
    deg2 = jnp.stack([deg, jnp.zeros((N, 2), jnp.float32)])
    return ew, deg2


def _edge_gcn(hc, ho, dinv, ew, src, dst):
    nc = dinv[src, 0] * ew[:, 0] * dinv[dst, 0]
    no = dinv[src, 1] * ew[:, 1] * dinv[dst, 1]
    outc = jnp.zeros((N, H), jnp.float32).at[dst].add(nc[:, None] * hc[src])
    outo = jnp.zeros((N, H), jnp.float32).at[dst].add(no[:, None] * ho[src])
    z = jnp.zeros((N, H), jnp.float32)
    return jnp.stack([outc, z]), jnp.stack([outo, z])


def R_layer_norm(x, g, b, eps=1e-5):
    m = x.mean(-1, keepdims=True)
    v = ((x - m) ** 2).mean(-1, keepdims=True)
    return (x - m) * jax.lax.rsqrt(v + eps) * g + b


def kernel(x, params, edge_index, batch, keypoints):
    p = params
    src, dst = edge_index[0], edge_index[1]
    src_i = src.astype(jnp.int32)
    dst_i = dst.astype(jnp.int32)

    def fold(Wn, gn, bn):
        Wp = p[gn][:, None] * p[Wn]
        cp = p[bn] @ p[Wn]
        return Wp, cp

    h = _stage0(x, p['bn_feat_g'], p['bn_feat_b'], p['conv_feat_W'],
                p['conv_feat_b'])

    for i in range(3):
        Wp = p['bn%d_g' % i][:, None] * p['gat%d_W' % i]
        cp = p['bn%d_b' % i] @ p['gat%d_W' % i]
        att_s = p['gat%d_as' % i]
        att_d = p['gat%d_ad' % i]
        Aall = jnp.zeros((H, 8), jnp.float32)
        for k in range(K):
            Aall = Aall.at[DH * k:DH * (k + 1), k].set(att_s[k])
            Aall = Aall.at[DH * k:DH * (k + 1), 4 + k].set(att_d[k])
        hw, aa = _gat_pre(h, Wp, cp, Aall)
        out2, den2 = _edge_gat_sc(hw, aa, src_i, dst_i)
        h = _gat_post(out2, den2, aa, hw, p['gat%d_b' % i])

    eb = p['edge_mlp_b']
    We = jnp.zeros((H, 8), jnp.float32)
    We = We.at[:, 0:2].set(p['edge_mlp_W'][:H])
    We = We.at[:, 2:4].set(p['edge_mlp_W'][H:])
    ebp = jnp.zeros((1, 8), jnp.float32).at[0, 0:2].set(eb)
    Wn = jnp.zeros((H, 8), jnp.float32).at[:, 0:2].set(p['node_mlp_W'])
    nbp = jnp.zeros((1, 8), jnp.float32).at[0, 0:2].set(p['node_mlp_b'])
    Wc, cc = fold('ctx_W', 'bnc_g', 'bnc_b')
    Wo, co = fold('obj_W', 'bno_g', 'bno_b')
    suv, hc, ho = _final(h, We, ebp, Wn, nbp, Wc, cc, Wo, co)

    ew, deg2 = _edge_att_deg_sc(suv, src_i, dst_i)
    dinv = _dinv(deg2)
    outc2 = _edge_gcn_sc(hc, dinv, ew, src_i, dst_i, 0)
    outo2 = _edge_gcn_sc(ho, dinv, ew, src_i, dst_i, 1)

    def packhead(pref):
        rows = [p['ln1%s_g' % pref].reshape(1, H), p['ln1%s_b' % pref].reshape(1, H),
                p['fc1%s_W' % pref], p['fc1%s_b' % pref].reshape(1, H),
                p['ln2%s_g' % pref].reshape(1, H), p['ln2%s_b' % pref].reshape(1, H),
                jnp.zeros((H, H), jnp.float32).at[:, :NC].set(p['fc2%s_W' % pref]),
                jnp.zeros((1, H), jnp.float32).at[0, :NC].set(p['fc2%s_b' % pref])]
        return jnp.concatenate(rows, axis=0)

    headpack = jnp.concatenate([packhead('c'), packhead('o')], axis=0)
    batch2d = batch.astype(jnp.int32).reshape(GRID, 1, BN)
    lc, lo, po = _pool_heads(outc2, outo2, hc, ho, dinv, batch2d,
                             p['ctx_b'], p['obj_b'], headpack)
    return (lc, lo, po)


def _final_jnp(h, We, ebp, Wn, nbp, Wc, cc, Wo, co):
    uv = h @ We + ebp
    su = uv[:, 0:1] - uv[:, 1:2]
    sv = uv[:, 2:3] - uv[:, 3:4]
    suv = jnp.concatenate([su, sv, jnp.zeros((N, 6), jnp.float32)], axis=1)
    na = jax.nn.softmax(h @ Wn[:, 0:2] + nbp[:, 0:2], axis=-1)
    hc = (na[:, 0:1] * h) @ Wc + cc.reshape(1, H)
    ho = (na[:, 1:2] * h) @ Wo + co.reshape(1, H)
    return suv, hc, ho


def _pool_heads_jnp(outc2, outo2, hc, ho, dinv, batch, cb, ob, p):
    xcn = jax.nn.relu(outc2[0] + outc2[1] + dinv[:, 0:1] ** 2 * hc + cb)
    xon = jax.nn.relu(outo2[0] + outo2[1] + dinv[:, 1:2] ** 2 * ho + ob)
    onehot = (batch[None, :] == jnp.arange(G)[:, None]).astype(jnp.float32)
    pc = onehot @ xcn
    po = onehot @ xon

    def ln(z, g, b):
        m = z.mean(-1, keepdims=True)
        v = ((z - m) ** 2).mean(-1, keepdims=True)
        return (z - m) * jax.lax.rsqrt(v + 1e-5) * g + b

    z = ln(pc, p['ln1c_g'], p['ln1c_b'])
    z = jax.nn.relu(z @ p['fc1c_W'] + p['fc1c_b'])
    z = ln(z, p['ln2c_g'], p['ln2c_b'])
    lc = z @ p['fc2c_W'] + p['fc2c_b']
    w = ln(po, p['ln1o_g'], p['ln1o_b'])
    w = jax.nn.relu(w @ p['fc1o_W'] + p['fc1o_b'])
    w = ln(w, p['ln2o_g'], p['ln2o_b'])
    lo = w @ p['fc2o_W'] + p['fc2o_b']
    return lc, lo, po


# speedup vs baseline: 47.5623x; 1.5127x over previous
import functools
import jax
import jax.numpy as jnp
from jax import lax
from jax.experimental import pallas as pl
from jax.experimental.pallas import tpu as pltpu
from jax.experimental.pallas import tpu_sc as plsc

N = 10000
E = 320000
H = 128
K = 4
DH = 32
G = 128
NC = 10
BN = 2000
GRID = N // BN


def _rows(i):
    return (i, 0)


def _full(i):
    return (0, 0)


def _k_stage0(x_ref, g_ref, b_ref, W_ref, c_ref, o_ref):
    x = x_ref[...]
    m = jnp.mean(x, axis=-1, keepdims=True)
    v = jnp.mean((x - m) ** 2, axis=-1, keepdims=True)
    ln = (x - m) * jax.lax.rsqrt(v + 1e-5) * g_ref[...] + b_ref[...]
    h = jnp.dot(ln, W_ref[...], preferred_element_type=jnp.float32) + c_ref[...]
    o_ref[...] = jnp.maximum(h, 0.0)


def _stage0(x, g, b, W, c):
    return pl.pallas_call(
        _k_stage0,
        grid=(GRID,),
        in_specs=[
            pl.BlockSpec((BN, H), _rows),
            pl.BlockSpec((1, H), _full),
            pl.BlockSpec((1, H), _full),
            pl.BlockSpec((H, H), _full),
            pl.BlockSpec((1, H), _full),
        ],
        out_specs=pl.BlockSpec((BN, H), _rows),
        out_shape=jax.ShapeDtypeStruct((N, H), jnp.float32),
    )(x, g.reshape(1, H), b.reshape(1, H), W, c.reshape(1, H))


def _k_gat_pre(h_ref, W_ref, c_ref, A_ref, hw_ref, aa_ref):
    hw = jnp.dot(h_ref[...], W_ref[...], preferred_element_type=jnp.float32) + c_ref[...]
    hw_ref[...] = hw
    aa_ref[...] = jnp.dot(hw, A_ref[...], preferred_element_type=jnp.float32)


def _gat_pre(h, Wp, cp, Aall):
    return pl.pallas_call(
        _k_gat_pre,
        grid=(GRID,),
        in_specs=[
            pl.BlockSpec((BN, H), _rows),
            pl.BlockSpec((H, H), _full),
            pl.BlockSpec((1, H), _full),
            pl.BlockSpec((H, 8), _full),
        ],
        out_specs=[
            pl.BlockSpec((BN, H), _rows),
            pl.BlockSpec((BN, 8), _rows),
        ],
        out_shape=[
            jax.ShapeDtypeStruct((N, H), jnp.float32),
            jax.ShapeDtypeStruct((N, 8), jnp.float32),
        ],
    )(h, Wp, cp.reshape(1, H), Aall)


def _k_gat_post(out2_ref, den2_ref, aa_ref, hw_ref, b_ref, o_ref):
    outs = out2_ref[0] + out2_ref[1]
    dens = den2_ref[0] + den2_ref[1]
    aa = aa_ref[...]
    tl = aa[:, 0:4] + aa[:, 4:8]
    al = jnp.maximum(tl, 0.0) + 0.2 * jnp.minimum(tl, 0.0)
    exl = jnp.exp(al)
    den = dens + exl
    hw = hw_ref[...]
    exb = jnp.repeat(exl, DH, axis=1)
    denb = jnp.repeat(den, DH, axis=1)
    o = (outs + exb * hw) / denb + b_ref[...]
    o_ref[...] = jnp.maximum(o, 0.0)


def _gat_post(out2, den2, aa, hw, bias):
    return pl.pallas_call(
        _k_gat_post,
        grid=(GRID,),
        in_specs=[
            pl.BlockSpec((2, BN, H), lambda i: (0, i, 0)),
            pl.BlockSpec((2, BN, K), lambda i: (0, i, 0)),
            pl.BlockSpec((BN, 8), _rows),
            pl.BlockSpec((BN, H), _rows),
            pl.BlockSpec((1, H), _full),
        ],
        out_specs=pl.BlockSpec((BN, H), _rows),
        out_shape=jax.ShapeDtypeStruct((N, H), jnp.float32),
    )(out2, den2, aa, hw, bias.reshape(1, H))


def _k_final(h_ref, We_ref, eb_ref, Wn_ref, nb_ref, Wc_ref, cc_ref,
             Wo_ref, co_ref, suv_ref, hc_ref, ho_ref):
    h = h_ref[...]
    uv = jnp.dot(h, We_ref[...], preferred_element_type=jnp.float32) + eb_ref[...]
    su = uv[:, 0:1] - uv[:, 1:2]
    sv = uv[:, 2:3] - uv[:, 3:4]
    suv = jnp.concatenate([su, sv, jnp.zeros_like(uv[:, 0:6])], axis=1)
    suv_ref[...] = suv
    nl = jnp.dot(h, Wn_ref[...], preferred_element_type=jnp.float32) + nb_ref[...]
    m = jnp.maximum(nl[:, 0:1], nl[:, 1:2])
    e0 = jnp.exp(nl[:, 0:1] - m)
    e1 = jnp.exp(nl[:, 1:2] - m)
    na0 = e0 / (e0 + e1)
    na1 = 1.0 - na0
    xc = na0 * h
    xo = na1 * h
    hc_ref[...] = jnp.dot(xc, Wc_ref[...], preferred_element_type=jnp.float32) + cc_ref[...]
    ho_ref[...] = jnp.dot(xo, Wo_ref[...], preferred_element_type=jnp.float32) + co_ref[...]


def _final(h, We, eb, Wn, nb, Wc, cc, Wo, co):
    return pl.pallas_call(
        _k_final,
        grid=(GRID,),
        in_specs=[
            pl.BlockSpec((BN, H), _rows),
            pl.BlockSpec((H, 8), _full),
            pl.BlockSpec((1, 8), _full),
            pl.BlockSpec((H, 8), _full),
            pl.BlockSpec((1, 8), _full),
            pl.BlockSpec((H, H), _full),
            pl.BlockSpec((1, H), _full),
            pl.BlockSpec((H, H), _full),
            pl.BlockSpec((1, H), _full),
        ],
        out_specs=[
            pl.BlockSpec((BN, 8), _rows),
            pl.BlockSpec((BN, H), _rows),
            pl.BlockSpec((BN, H), _rows),
        ],
        out_shape=[
            jax.ShapeDtypeStruct((N, 8), jnp.float32),
            jax.ShapeDtypeStruct((N, H), jnp.float32),
            jax.ShapeDtypeStruct((N, H), jnp.float32),
        ],
    )(h, We, eb, Wn, nb, Wc, cc.reshape(1, H), Wo, co.reshape(1, H))


def _k_dinv(deg2_ref, o_ref):
    d = 1.0 + deg2_ref[0] + deg2_ref[1]
    o_ref[...] = jax.lax.rsqrt(d)


def _dinv(deg2):
    return pl.pallas_call(
        _k_dinv,
        grid=(GRID,),
        in_specs=[pl.BlockSpec((2, BN, 2), lambda i: (0, i, 0))],
        out_specs=pl.BlockSpec((BN, 2), _rows),
        out_shape=jax.ShapeDtypeStruct((N, 2), jnp.float32),
    )(deg2)


def _k_pool(outc2_ref, outo2_ref, hc_ref, ho_ref, dinv_ref, batch_ref,
            cb_ref, ob_ref, hp_ref,
            lc_ref, lo_ref, po_ref, pc_acc):
    i = pl.program_id(0)

    @pl.when(i == 0)
    def _init():
        pc_acc[...] = jnp.zeros_like(pc_acc)
        po_ref[...] = jnp.zeros_like(po_ref)

    dinv = dinv_ref[...]
    d2c = dinv[:, 0:1] * dinv[:, 0:1]
    d2o = dinv[:, 1:2] * dinv[:, 1:2]
    xcn = jnp.maximum(outc2_ref[0] + outc2_ref[1] + d2c * hc_ref[...] + cb_ref[...], 0.0)
    xon = jnp.maximum(outo2_ref[0] + outo2_ref[1] + d2o * ho_ref[...] + ob_ref[...], 0.0)
    gids = jax.lax.broadcasted_iota(jnp.int32, (G, BN), 0)
    onehot = (gids == batch_ref[0]).astype(jnp.float32)
    pc_acc[...] += jnp.dot(onehot, xcn, preferred_element_type=jnp.float32)
    po_ref[...] += jnp.dot(onehot, xon, preferred_element_type=jnp.float32)

    @pl.when(i == GRID - 1)
    def _heads():
        hp = hp_ref[...]

        def ln(z, g, b):
            m = jnp.mean(z, axis=-1, keepdims=True)
            v = jnp.mean((z - m) ** 2, axis=-1, keepdims=True)
            return (z - m) * jax.lax.rsqrt(v + 1e-5) * g + b

        def head(pool, base):
            z = ln(pool, hp[base + 0:base + 1], hp[base + 1:base + 2])
            z = jnp.maximum(jnp.dot(z, hp[base + 2:base + 130],
                                    preferred_element_type=jnp.float32)
                            + hp[base + 130:base + 131], 0.0)
            z = ln(z, hp[base + 131:base + 132], hp[base + 132:base + 133])
            lg = jnp.dot(z, hp[base + 133:base + 261],
                         preferred_element_type=jnp.float32)[:, :NC] \
                + hp[base + 261:base + 262, :NC]
            return lg

        lc_ref[...] = head(pc_acc[...], 0)
        lo_ref[...] = head(po_ref[...], 262)


def _pool_heads(outc2, outo2, hc, ho, dinv, batch2d, cb, ob, headpack):
    return pl.pallas_call(
        _k_pool,
        grid=(GRID,),
        in_specs=[
            pl.BlockSpec((2, BN, H), lambda i: (0, i, 0)),
            pl.BlockSpec((2, BN, H), lambda i: (0, i, 0)),
            pl.BlockSpec((BN, H), _rows),
            pl.BlockSpec((BN, H), _rows),
            pl.BlockSpec((BN, 2), _rows),
            pl.BlockSpec((1, 1, BN), lambda i: (i, 0, 0)),
            pl.BlockSpec((1, H), _full),
            pl.BlockSpec((1, H), _full),
            pl.BlockSpec((524, H), _full),
        ],
        out_specs=[
            pl.BlockSpec((G, NC), _full),
            pl.BlockSpec((G, NC), _full),
            pl.BlockSpec((G, H), _full),
        ],
        out_shape=[
            jax.ShapeDtypeStruct((G, NC), jnp.float32),
            jax.ShapeDtypeStruct((G, NC), jnp.float32),
            jax.ShapeDtypeStruct((G, H), jnp.float32),
        ],
        scratch_shapes=[pltpu.VMEM((G, H), jnp.float32)],
    )(outc2, outo2, hc, ho, dinv, batch2d, cb.reshape(1, H), ob.reshape(1, H),
      headpack)


SC_C = 2
SC_T = 16
NW = SC_C * SC_T
EW = E // NW
CH = 256
TCHUNK = E // CH
NCHUNK = TCHUNK // NW
CREM = TCHUNK - NCHUNK * NW
NPAD = 10240
ZROWS = NPAD // SC_T
ZC = 128
ZN = ZROWS // ZC
OC = 80

_PE = tuple([e for e in range(4) for _ in range(4)])
_HM = tuple(list(range(4)) * 4)


def _sc_gat_body(hw_hbm, aa_hbm, src_hbm, dst_hbm, out_hbm, den_hbm,
                 src_v, dst_v, si_v, ddi_v, di_v, ex_v, asg_v, adg_v,
                 rows_v, out_sh, den_sh, sem):
    c = lax.axis_index("c")
    s = lax.axis_index("s")
    w = s * SC_C + c
    iota = lax.iota(jnp.int32, 16)
    pe = iota >> 2
    hm = iota & 3
    zero = jnp.full((16,), 0.0, jnp.float32)

    def _zr(j, _):
        for t in range(H // 16):
            rows_v[j, pl.ds(16 * t, 16)] = zero
        return 0
    lax.fori_loop(0, CH, _zr, 0)

    def _ze(q, _):
        ex_v[pl.ds(16 * q, 16)] = zero
        return 0
    lax.fori_loop(0, CH * K // 16, _ze, 0)

    for i in range(ZN):
        start = s * ZROWS + i * ZC
        pltpu.sync_copy(rows_v.at[pl.ds(0, ZC)], out_sh.at[pl.ds(start, ZC)])
        pltpu.sync_copy(ex_v.at[pl.ds(0, K * ZC)],
                        den_sh.at[pl.ds(K * start, K * ZC)])
    plsc.subcore_barrier()

    def _chunk(g, _):
        base = (w + NW * g) * CH
        pltpu.sync_copy(src_hbm.at[pl.ds(base, CH)], src_v)
        pltpu.sync_copy(dst_hbm.at[pl.ds(base, CH)], dst_v)
        hrows = pltpu.async_copy(hw_hbm.at[src_v], rows_v, sem)

        def _idx4(q, _):
            idx_e = pe + 4 * q
            srcs = plsc.load_gather(src_v, [idx_e])
            dsts = plsc.load_gather(dst_v, [idx_e])
            si_v[pl.ds(16 * q, 16)] = srcs * 8 + hm
            ddi_v[pl.ds(16 * q, 16)] = dsts * 8 + (hm + 4)
            di_v[pl.ds(16 * q, 16)] = dsts * K + hm
            return 0
        lax.fori_loop(0, CH // 4, _idx4, 0)

        pltpu.sync_copy(aa_hbm.at[si_v], asg_v)
        pltpu.sync_copy(aa_hbm.at[ddi_v], adg_v)

        def _ex16(q, _):
            t = asg_v[pl.ds(16 * q, 16)] + adg_v[pl.ds(16 * q, 16)]
            alpha = jnp.maximum(t, 0.0) + 0.2 * jnp.minimum(t, 0.0)
            ex_v[pl.ds(16 * q, 16)] = jnp.exp(alpha)
            return 0
        lax.fori_loop(0, CH * K // 16, _ex16, 0)

        hrows.wait()

        def _scale4(q, _):
            exv = ex_v[pl.ds(16 * q, 16)]
            for j in range(4):
                e = 4 * q + j
                for k in range(K):
                    sel = jnp.full((16,), 4 * j + k, jnp.int32)
                    sc = exv.at[sel].get(mode="promise_in_bounds")
                    for t2 in range(2):
                        col = k * DH + 16 * t2
                        rows_v[e, pl.ds(col, 16)] = rows_v[e, pl.ds(col, 16)] * sc
            return 0
        lax.fori_loop(0, CH // 4, _scale4, 0)

        pltpu.sync_copy(ex_v, den_sh.at[di_v], add=True)
        pltpu.sync_copy(rows_v, out_sh.at[dst_v], add=True)
        return 0

    lax.fori_loop(0, NCHUNK, _chunk, 0)

    @pl.when(w < CREM)
    def _extra():
        _chunk(NCHUNK, 0)
    plsc.subcore_barrier()

    for i in range(ZROWS // OC):
        start = s * ZROWS + i * OC

        @pl.when(start < N)
        def _copy_out():
            pltpu.sync_copy(out_sh.at[pl.ds(start, OC)],
                            rows_v.at[pl.ds(0, OC)])
            pltpu.sync_copy(rows_v.at[pl.ds(0, OC)],
                            out_hbm.at[c, pl.ds(start, OC)])
            pltpu.sync_copy(den_sh.at[pl.ds(K * start, K * OC)],
                            ex_v.at[pl.ds(0, K * OC)])
            pltpu.sync_copy(ex_v.at[pl.ds(0, K * OC)],
                            den_hbm.at[pl.ds(c * (N * K) + K * start, K * OC)])


def _edge_gat_sc(hw, aa, src, dst):
    mesh = plsc.VectorSubcoreMesh(core_axis_name="c", subcore_axis_name="s")
    f = pl.kernel(
        _sc_gat_body,
        mesh=mesh,
        compiler_params=pltpu.CompilerParams(needs_layout_passes=False),
        out_type=[
            jax.ShapeDtypeStruct((SC_C, N, H), jnp.float32),
            jax.ShapeDtypeStruct((SC_C * N * K,), jnp.float32),
        ],
        scratch_types=[
            pltpu.VMEM((CH,), jnp.int32),
            pltpu.VMEM((CH,), jnp.int32),
            pltpu.VMEM((CH * K,), jnp.int32),
            pltpu.VMEM((CH * K,), jnp.int32),
            pltpu.VMEM((CH * K,), jnp.int32),
            pltpu.VMEM((CH * K,), jnp.float32),
            pltpu.VMEM((CH * K,), jnp.float32),
            pltpu.VMEM((CH * K,), jnp.float32),
            pltpu.VMEM((CH, H), jnp.float32),
            pltpu.VMEM_SHARED((NPAD, H), jnp.float32),
            pltpu.VMEM_SHARED((NPAD * K,), jnp.float32),
            pltpu.SemaphoreType.DMA,
        ],
    )
    out2, denf = f(hw, aa.reshape(N * 8), src, dst)
    return out2, denf.reshape(SC_C, N, K)


def _sc_att_body(suv_hbm, src_hbm, dst_hbm, ew_hbm, deg_hbm,
                 src_v, dst_v, si_v, dvi_v, di_v, sug_v, svg_v, ew_v,
                 deg_sh, sem):
    c = lax.axis_index("c")
    s = lax.axis_index("s")
    w = s * SC_C + c
    iota = lax.iota(jnp.int32, 16)
    pe2 = iota >> 1
    m2 = iota & 1
    zero = jnp.full((16,), 0.0, jnp.float32)

    def _zw(q, _):
        ew_v[pl.ds(16 * q, 16)] = zero
        return 0
    lax.fori_loop(0, CH * 2 // 16, _zw, 0)
    for i in range(ZN):
        start = s * ZROWS + i * ZC
        pltpu.sync_copy(ew_v.at[pl.ds(0, 2 * ZC)],
                        deg_sh.at[pl.ds(2 * start, 2 * ZC)])
    plsc.subcore_barrier()

    def _chunk(g, _):
        base = (w + NW * g) * CH
        pltpu.sync_copy(src_hbm.at[pl.ds(base, CH)], src_v)
        pltpu.sync_copy(dst_hbm.at[pl.ds(base, CH)], dst_v)

        def _idx8(q, _):
            idx_e = pe2 + 8 * q
            srcs = plsc.load_gather(src_v, [idx_e])
            dsts = plsc.load_gather(dst_v, [idx_e])
            si_v[pl.ds(16 * q, 16)] = srcs * 8
            dvi_v[pl.ds(16 * q, 16)] = dsts * 8 + 1
            di_v[pl.ds(16 * q, 16)] = srcs * 2 + m2
            return 0
        lax.fori_loop(0, CH * 2 // 16, _idx8, 0)

        pltpu.sync_copy(suv_hbm.at[si_v], sug_v)
        pltpu.sync_copy(suv_hbm.at[dvi_v], svg_v)

        def _cw(q, _):
            t = sug_v[pl.ds(16 * q, 16)] + svg_v[pl.ds(16 * q, 16)]
            ewc = 1.0 / (1.0 + jnp.exp(-t))
            ew_v[pl.ds(16 * q, 16)] = jnp.where(m2 == 0, ewc, 1.0 - ewc)
            return 0
        lax.fori_loop(0, CH * 2 // 16, _cw, 0)

        pltpu.sync_copy(ew_v, ew_hbm.at[pl.ds(2 * base, 2 * CH)])
        pltpu.sync_copy(ew_v, deg_sh.at[di_v], add=True)
        return 0

    lax.fori_loop(0, NCHUNK, _chunk, 0)

    @pl.when(w < CREM)
    def _extra():
        _chunk(NCHUNK, 0)
    plsc.subcore_barrier()
    for i in range(ZROWS // OC):
        start = s * ZROWS + i * OC

        @pl.when(start < N)
        def _copy_out():
            pltpu.sync_copy(deg_sh.at[pl.ds(2 * start, 2 * OC)],
                            ew_v.at[pl.ds(0, 2 * OC)])
            pltpu.sync_copy(ew_v.at[pl.ds(0, 2 * OC)],
                            deg_hbm.at[pl.ds(c * (N * 2) + 2 * start, 2 * OC)])


def _edge_att_deg_sc(suv, src, dst):
    mesh = plsc.VectorSubcoreMesh(core_axis_name="c", subcore_axis_name="s")
    f = pl.kernel(
        _sc_att_body,
        mesh=mesh,
        compiler_params=pltpu.CompilerParams(needs_layout_passes=False),
        out_type=[
            jax.ShapeDtypeStruct((E * 2,), jnp.float32),
            jax.ShapeDtypeStruct((SC_C * N * 2,), jnp.float32),
        ],
        scratch_types=[
            pltpu.VMEM((CH,), jnp.int32),
            pltpu.VMEM((CH,), jnp.int32),
            pltpu.VMEM((CH * 2,), jnp.int32),
            pltpu.VMEM((CH * 2,), jnp.int32),
            pltpu.VMEM((CH * 2,), jnp.int32),
            pltpu.VMEM((CH * 2,), jnp.float32),
            pltpu.VMEM((CH * 2,), jnp.float32),
            pltpu.VMEM((CH * 2,), jnp.float32),
            pltpu.VMEM_SHARED((NPAD * 2,), jnp.float32),
            pltpu.SemaphoreType.DMA,
        ],
    )
    ew, degf = f(suv.reshape(N * 8), src, dst)
    return ew, degf.reshape(SC_C, N, 2)


def _make_gcn_body(off):
    def body(h_hbm, dinv_hbm, ew_hbm, src_hbm, dst_hbm, out_hbm,
             src_v, dst_v, si_v, dvi_v, ewi_v, ds_v, dd_v, ewv_v, nm_v,
             rows_v, out_sh, sem):
        c = lax.axis_index("c")
        s = lax.axis_index("s")
        w = s * SC_C + c
        iota = lax.iota(jnp.int32, 16)
        zero = jnp.full((16,), 0.0, jnp.float32)

        def _zr(j, _):
            for t in range(H // 16):
                rows_v[j, pl.ds(16 * t, 16)] = zero
            return 0
        lax.fori_loop(0, CH, _zr, 0)
        for i in range(ZN):
            start = s * ZROWS + i * ZC
            pltpu.sync_copy(rows_v.at[pl.ds(0, ZC)],
                            out_sh.at[pl.ds(start, ZC)])
        plsc.subcore_barrier()

        def _chunk(g, _):
            base = (w + NW * g) * CH
            pltpu.sync_copy(src_hbm.at[pl.ds(base, CH)], src_v)
            pltpu.sync_copy(dst_hbm.at[pl.ds(base, CH)], dst_v)
            hrows = pltpu.async_copy(h_hbm.at[src_v], rows_v, sem)

            def _idx16(q, _):
                idx_e = iota + 16 * q
                srcs = plsc.load_gather(src_v, [idx_e])
                dsts = plsc.load_gather(dst_v, [idx_e])
                si_v[pl.ds(16 * q, 16)] = srcs * 2 + off
                dvi_v[pl.ds(16 * q, 16)] = dsts * 2 + off
                ewi_v[pl.ds(16 * q, 16)] = (base + idx_e) * 2 + off
                return 0
            lax.fori_loop(0, CH // 16, _idx16, 0)

            pltpu.sync_copy(dinv_hbm.at[si_v], ds_v)
            pltpu.sync_copy(dinv_hbm.at[dvi_v], dd_v)
            pltpu.sync_copy(ew_hbm.at[ewi_v], ewv_v)

            def _nm(q, _):
                sl = pl.ds(16 * q, 16)
                nm_v[sl] = ds_v[sl] * ewv_v[sl] * dd_v[sl]
                return 0
            lax.fori_loop(0, CH // 16, _nm, 0)

            hrows.wait()

            def _scale(q, _):
                nv = nm_v[pl.ds(16 * q, 16)]
                for j in range(16):
                    e = 16 * q + j
                    sel = jnp.full((16,), j, jnp.int32)
                    sc = nv.at[sel].get(mode="promise_in_bounds")
                    for t2 in range(H // 16):
                        col = 16 * t2
                        rows_v[e, pl.ds(col, 16)] = rows_v[e, pl.ds(col, 16)] * sc
                return 0
            lax.fori_loop(0, CH // 16, _scale, 0)

            pltpu.sync_copy(rows_v, out_sh.at[dst_v], add=True)
            return 0

        lax.fori_loop(0, NCHUNK, _chunk, 0)

        @pl.when(w < CREM)
        def _extra():
            _chunk(NCHUNK, 0)
        plsc.subcore_barrier()
        for i in range(ZROWS // OC):
            start = s * ZROWS + i * OC

            @pl.when(start < N)
            def _copy_out():
                pltpu.sync_copy(out_sh.at[pl.ds(start, OC)],
                                rows_v.at[pl.ds(0, OC)])
                pltpu.sync_copy(rows_v.at[pl.ds(0, OC)],
                                out_hbm.at[c, pl.ds(start, OC)])
    return body


def _edge_gcn_sc(h, dinv, ew, src, dst, off):
    mesh = plsc.VectorSubcoreMesh(core_axis_name="c", subcore_axis_name="s")
    f = pl.kernel(
        _make_gcn_body(off),
        mesh=mesh,
        compiler_params=pltpu.CompilerParams(needs_layout_passes=False),
        out_type=jax.ShapeDtypeStruct((SC_C, N, H), jnp.float32),
        scratch_types=[
            pltpu.VMEM((CH,), jnp.int32),
            pltpu.VMEM((CH,), jnp.int32),
            pltpu.VMEM((CH,), jnp.int32),
            pltpu.VMEM((CH,), jnp.int32),
            pltpu.VMEM((CH,), jnp.int32),
            pltpu.VMEM((CH,), jnp.float32),
            pltpu.VMEM((CH,), jnp.float32),
            pltpu.VMEM((CH,), jnp.float32),
            pltpu.VMEM((CH,), jnp.float32),
            pltpu.VMEM((CH, H), jnp.float32),
            pltpu.VMEM_SHARED((NPAD, H), jnp.float32),
            pltpu.SemaphoreType.DMA,
        ],
    )
    return f(h, dinv.reshape(N * 2), ew, src, dst)


def _edge_gat(hw, aa, src, dst):
    t = aa[src, 0:4] + aa[dst, 4:8]
    alpha = jnp.maximum(t, 0) + 0.2 * jnp.minimum(t, 0)
    ex = jnp.exp(alpha)
    den = jnp.zeros((N, K), jnp.float32).at[dst].add(ex)
    hw4 = hw.reshape(N, K, DH)
    out = jnp.zeros((N, K, DH), jnp.float32).at[dst].add(ex[:, :, None] * hw4[src])
    out2 = jnp.stack([out.reshape(N, H), jnp.zeros((N, H), jnp.float32)])
    den2 = jnp.stack([den, jnp.zeros((N, K), jnp.float32)])
    return out2, den2


def _edge_att_deg(suv, src, dst):
    t = suv[src, 0] + suv[dst, 1]
    ewc = jax.nn.sigmoid(t)
    ewo = 1.0 - ewc
    ew = jnp.stack([ewc, ewo], axis=1)
    deg = jnp.zeros((N, 2), jnp.float32).at[src].add(ew)
    deg2 = jnp.stack([deg, jnp.zeros((N, 2), jnp.float32)])
    return ew, deg2


def _edge_gcn(hc, ho, dinv, ew, src, dst):
    nc = dinv[src, 0] * ew[:, 0] * dinv[dst, 0]
    no = dinv[src, 1] * ew[:, 1] * dinv[dst, 1]
    outc = jnp.zeros((N, H), jnp.float32).at[dst].add(nc[:, None] * hc[src])
    outo = jnp.zeros((N, H), jnp.float32).at[dst].add(no[:, None] * ho[src])
    z = jnp.zeros((N, H), jnp.float32)
    return jnp.stack([outc, z]), jnp.stack([outo, z])


def R_layer_norm(x, g, b, eps=1e-5):
    m = x.mean(-1, keepdims=True)
    v = ((x - m) ** 2).mean(-1, keepdims=True)
    return (x - m) * jax.lax.rsqrt(v + eps) * g + b


def kernel(x, params, edge_index, batch, keypoints):
    p = params
    src, dst = edge_index[0], edge_index[1]
    src_i = src.astype(jnp.int32)
    dst_i = dst.astype(jnp.int32)

    def fold(Wn, gn, bn):
        Wp = p[gn][:, None] * p[Wn]
        cp = p[bn] @ p[Wn]
        return Wp, cp

    h = _stage0(x, p['bn_feat_g'], p['bn_feat_b'], p['conv_feat_W'],
                p['conv_feat_b'])

    for i in range(3):
        Wp = p['bn%d_g' % i][:, None] * p['gat%d_W' % i]
        cp = p['bn%d_b' % i] @ p['gat%d_W' % i]
        att_s = p['gat%d_as' % i]
        att_d = p['gat%d_ad' % i]
        Aall = jnp.zeros((H, 8), jnp.float32)
        for k in range(K):
            Aall = Aall.at[DH * k:DH * (k + 1), k].set(att_s[k])
            Aall = Aall.at[DH * k:DH * (k + 1), 4 + k].set(att_d[k])
        hw, aa = _gat_pre(h, Wp, cp, Aall)
        out2, den2 = _edge_gat_sc(hw, aa, src_i, dst_i)
        h = _gat_post(out2, den2, aa, hw, p['gat%d_b' % i])

    eb = p['edge_mlp_b']
    We = jnp.zeros((H, 8), jnp.float32)
    We = We.at[:, 0:2].set(p['edge_mlp_W'][:H])
    We = We.at[:, 2:4].set(p['edge_mlp_W'][H:])
    ebp = jnp.zeros((1, 8), jnp.float32).at[0, 0:2].set(eb)
    Wn = jnp.zeros((H, 8), jnp.float32).at[:, 0:2].set(p['node_mlp_W'])
    nbp = jnp.zeros((1, 8), jnp.float32).at[0, 0:2].set(p['node_mlp_b'])
    Wc, cc = fold('ctx_W', 'bnc_g', 'bnc_b')
    Wo, co = fold('obj_W', 'bno_g', 'bno_b')
    suv, hc, ho = _final(h, We, ebp, Wn, nbp, Wc, cc, Wo, co)

    ew, deg2 = _edge_att_deg_sc(suv, src_i, dst_i)
    dinv = _dinv(deg2)
    outc2 = _edge_gcn_sc(hc, dinv, ew, src_i, dst_i, 0)
    outo2 = _edge_gcn_sc(ho, dinv, ew, src_i, dst_i, 1)

    def packhead(pref):
        rows = [p['ln1%s_g' % pref].reshape(1, H), p['ln1%s_b' % pref].reshape(1, H),
                p['fc1%s_W' % pref], p['fc1%s_b' % pref].reshape(1, H),
                p['ln2%s_g' % pref].reshape(1, H), p['ln2%s_b' % pref].reshape(1, H),
                jnp.zeros((H, H), jnp.float32).at[:, :NC].set(p['fc2%s_W' % pref]),
                jnp.zeros((1, H), jnp.float32).at[0, :NC].set(p['fc2%s_b' % pref])]
        return jnp.concatenate(rows, axis=0)

    headpack = jnp.concatenate([packhead('c'), packhead('o')], axis=0)
    batch2d = batch.astype(jnp.int32).reshape(GRID, 1, BN)
    lc, lo, po = _pool_heads(outc2, outo2, hc, ho, dinv, batch2d,
                             p['ctx_b'], p['obj_b'], headpack)
    return (lc, lo, po)


def _final_jnp(h, We, ebp, Wn, nbp, Wc, cc, Wo, co):
    uv = h @ We + ebp
    su = uv[:, 0:1] - uv[:, 1:2]
    sv = uv[:, 2:3] - uv[:, 3:4]
    suv = jnp.concatenate([su, sv, jnp.zeros((N, 6), jnp.float32)], axis=1)
    na = jax.nn.softmax(h @ Wn[:, 0:2] + nbp[:, 0:2], axis=-1)
    hc = (na[:, 0:1] * h) @ Wc + cc.reshape(1, H)
    ho = (na[:, 1:2] * h) @ Wo + co.reshape(1, H)
    return suv, hc, ho


def _pool_heads_jnp(outc2, outo2, hc, ho, dinv, batch, cb, ob, p):
    xcn = jax.nn.relu(outc2[0] + outc2[1] + dinv[:, 0:1] ** 2 * hc + cb)
    xon = jax.nn.relu(outo2[0] + outo2[1] + dinv[:, 1:2] ** 2 * ho + ob)
    onehot = (batch[None, :] == jnp.arange(G)[:, None]).astype(jnp.float32)
    pc = onehot @ xcn
    po = onehot @ xon

    def ln(z, g, b):
        m = z.mean(-1, keepdims=True)
        v = ((z - m) ** 2).mean(-1, keepdims=True)
        return (z - m) * jax.lax.rsqrt(v + 1e-5) * g + b

    z = ln(pc, p['ln1c_g'], p['ln1c_b'])
    z = jax.nn.relu(z @ p['fc1c_W'] + p['fc1c_b'])
    z = ln(z, p['ln2c_g'], p['ln2c_b'])
    lc = z @ p['fc2c_W'] + p['fc2c_b']
    w = ln(po, p['ln1o_g'], p['ln1o_b'])
    w = jax.nn.relu(w @ p['fc1o_W'] + p['fc1o_b'])
    w = ln(w, p['ln2o_g'], p['ln2o_b'])
    lo = w @ p['fc2o_W'] + p['fc2o_b']
    return lc, lo, po


# speedup vs baseline: 47.9674x; 1.0085x over previous
import functools
import jax
import jax.numpy as jnp
from jax import lax
from jax.experimental import pallas as pl
from jax.experimental.pallas import tpu as pltpu
from jax.experimental.pallas import tpu_sc as plsc

N = 10000
E = 320000
H = 128
K = 4
DH = 32
G = 128
NC = 10
BN = 2000
GRID = N // BN


def _rows(i):
    return (i, 0)


def _full(i):
    return (0, 0)


def _k_stage0(x_ref, g_ref, b_ref, W_ref, c_ref, o_ref):
    x = x_ref[...]
    m = jnp.mean(x, axis=-1, keepdims=True)
    v = jnp.mean((x - m) ** 2, axis=-1, keepdims=True)
    ln = (x - m) * jax.lax.rsqrt(v + 1e-5) * g_ref[...] + b_ref[...]
    h = jnp.dot(ln, W_ref[...], preferred_element_type=jnp.float32) + c_ref[...]
    o_ref[...] = jnp.maximum(h, 0.0)


def _stage0(x, g, b, W, c):
    return pl.pallas_call(
        _k_stage0,
        grid=(GRID,),
        in_specs=[
            pl.BlockSpec((BN, H), _rows),
            pl.BlockSpec((1, H), _full),
            pl.BlockSpec((1, H), _full),
            pl.BlockSpec((H, H), _full),
            pl.BlockSpec((1, H), _full),
        ],
        out_specs=pl.BlockSpec((BN, H), _rows),
        out_shape=jax.ShapeDtypeStruct((N, H), jnp.float32),
    )(x, g.reshape(1, H), b.reshape(1, H), W, c.reshape(1, H))


def _k_gat_pre(h_ref, W_ref, c_ref, A_ref, hw_ref, aa_ref):
    hw = jnp.dot(h_ref[...], W_ref[...], preferred_element_type=jnp.float32) + c_ref[...]
    hw_ref[...] = hw
    aa_ref[...] = jnp.dot(hw, A_ref[...], preferred_element_type=jnp.float32)


def _gat_pre(h, Wp, cp, Aall):
    return pl.pallas_call(
        _k_gat_pre,
        grid=(GRID,),
        in_specs=[
            pl.BlockSpec((BN, H), _rows),
            pl.BlockSpec((H, H), _full),
            pl.BlockSpec((1, H), _full),
            pl.BlockSpec((H, 8), _full),
        ],
        out_specs=[
            pl.BlockSpec((BN, H), _rows),
            pl.BlockSpec((BN, 8), _rows),
        ],
        out_shape=[
            jax.ShapeDtypeStruct((N, H), jnp.float32),
            jax.ShapeDtypeStruct((N, 8), jnp.float32),
        ],
    )(h, Wp, cp.reshape(1, H), Aall)


def _k_gat_post(out2_ref, den2_ref, aa_ref, hw_ref, b_ref, o_ref):
    outs = out2_ref[0] + out2_ref[1]
    dens = den2_ref[0] + den2_ref[1]
    aa = aa_ref[...]
    tl = aa[:, 0:4] + aa[:, 4:8]
    al = jnp.maximum(tl, 0.0) + 0.2 * jnp.minimum(tl, 0.0)
    exl = jnp.exp(al)
    den = dens + exl
    hw = hw_ref[...]
    exb = jnp.repeat(exl, DH, axis=1)
    denb = jnp.repeat(den, DH, axis=1)
    o = (outs + exb * hw) / denb + b_ref[...]
    o_ref[...] = jnp.maximum(o, 0.0)


def _gat_post(out2, den2, aa, hw, bias):
    return pl.pallas_call(
        _k_gat_post,
        grid=(GRID,),
        in_specs=[
            pl.BlockSpec((2, BN, H), lambda i: (0, i, 0)),
            pl.BlockSpec((2, BN, K), lambda i: (0, i, 0)),
            pl.BlockSpec((BN, 8), _rows),
            pl.BlockSpec((BN, H), _rows),
            pl.BlockSpec((1, H), _full),
        ],
        out_specs=pl.BlockSpec((BN, H), _rows),
        out_shape=jax.ShapeDtypeStruct((N, H), jnp.float32),
    )(out2, den2, aa, hw, bias.reshape(1, H))


def _k_final(h_ref, We_ref, eb_ref, Wn_ref, nb_ref, Wc_ref, cc_ref,
             Wo_ref, co_ref, suv_ref, hc_ref, ho_ref):
    h = h_ref[...]
    uv = jnp.dot(h, We_ref[...], preferred_element_type=jnp.float32) + eb_ref[...]
    su = uv[:, 0:1] - uv[:, 1:2]
    sv = uv[:, 2:3] - uv[:, 3:4]
    suv = jnp.concatenate([su, sv, jnp.zeros_like(uv[:, 0:6])], axis=1)
    suv_ref[...] = suv
    nl = jnp.dot(h, Wn_ref[...], preferred_element_type=jnp.float32) + nb_ref[...]
    m = jnp.maximum(nl[:, 0:1], nl[:, 1:2])
    e0 = jnp.exp(nl[:, 0:1] - m)
    e1 = jnp.exp(nl[:, 1:2] - m)
    na0 = e0 / (e0 + e1)
    na1 = 1.0 - na0
    xc = na0 * h
    xo = na1 * h
    hc_ref[...] = jnp.dot(xc, Wc_ref[...], preferred_element_type=jnp.float32) + cc_ref[...]
    ho_ref[...] = jnp.dot(xo, Wo_ref[...], preferred_element_type=jnp.float32) + co_ref[...]


def _final(h, We, eb, Wn, nb, Wc, cc, Wo, co):
    return pl.pallas_call(
        _k_final,
        grid=(GRID,),
        in_specs=[
            pl.BlockSpec((BN, H), _rows),
            pl.BlockSpec((H, 8), _full),
            pl.BlockSpec((1, 8), _full),
            pl.BlockSpec((H, 8), _full),
            pl.BlockSpec((1, 8), _full),
            pl.BlockSpec((H, H), _full),
            pl.BlockSpec((1, H), _full),
            pl.BlockSpec((H, H), _full),
            pl.BlockSpec((1, H), _full),
        ],
        out_specs=[
            pl.BlockSpec((BN, 8), _rows),
            pl.BlockSpec((BN, H), _rows),
            pl.BlockSpec((BN, H), _rows),
        ],
        out_shape=[
            jax.ShapeDtypeStruct((N, 8), jnp.float32),
            jax.ShapeDtypeStruct((N, H), jnp.float32),
            jax.ShapeDtypeStruct((N, H), jnp.float32),
        ],
    )(h, We, eb, Wn, nb, Wc, cc.reshape(1, H), Wo, co.reshape(1, H))


def _k_dinv(deg2_ref, o_ref):
    d = 1.0 + deg2_ref[0] + deg2_ref[1]
    o_ref[...] = jax.lax.rsqrt(d)


def _dinv(deg2):
    return pl.pallas_call(
        _k_dinv,
        grid=(GRID,),
        in_specs=[pl.BlockSpec((2, BN, 2), lambda i: (0, i, 0))],
        out_specs=pl.BlockSpec((BN, 2), _rows),
        out_shape=jax.ShapeDtypeStruct((N, 2), jnp.float32),
    )(deg2)


def _k_pool(outc2_ref, outo2_ref, hc_ref, ho_ref, dinv_ref, batch_ref,
            cb_ref, ob_ref, hp_ref,
            lc_ref, lo_ref, po_ref, pc_acc):
    i = pl.program_id(0)

    @pl.when(i == 0)
    def _init():
        pc_acc[...] = jnp.zeros_like(pc_acc)
        po_ref[...] = jnp.zeros_like(po_ref)

    dinv = dinv_ref[...]
    d2c = dinv[:, 0:1] * dinv[:, 0:1]
    d2o = dinv[:, 1:2] * dinv[:, 1:2]
    xcn = jnp.maximum(outc2_ref[0] + outc2_ref[1] + d2c * hc_ref[...] + cb_ref[...], 0.0)
    xon = jnp.maximum(outo2_ref[0] + outo2_ref[1] + d2o * ho_ref[...] + ob_ref[...], 0.0)
    gids = jax.lax.broadcasted_iota(jnp.int32, (G, BN), 0)
    onehot = (gids == batch_ref[0]).astype(jnp.float32)
    pc_acc[...] += jnp.dot(onehot, xcn, preferred_element_type=jnp.float32)
    po_ref[...] += jnp.dot(onehot, xon, preferred_element_type=jnp.float32)

    @pl.when(i == GRID - 1)
    def _heads():
        hp = hp_ref[...]

        def ln(z, g, b):
            m = jnp.mean(z, axis=-1, keepdims=True)
            v = jnp.mean((z - m) ** 2, axis=-1, keepdims=True)
            return (z - m) * jax.lax.rsqrt(v + 1e-5) * g + b

        def head(pool, base):
            z = ln(pool, hp[base + 0:base + 1], hp[base + 1:base + 2])
            z = jnp.maximum(jnp.dot(z, hp[base + 2:base + 130],
                                    preferred_element_type=jnp.float32)
                            + hp[base + 130:base + 131], 0.0)
            z = ln(z, hp[base + 131:base + 132], hp[base + 132:base + 133])
            lg = jnp.dot(z, hp[base + 133:base + 261],
                         preferred_element_type=jnp.float32)[:, :NC] \
                + hp[base + 261:base + 262, :NC]
            return lg

        lc_ref[...] = head(pc_acc[...], 0)
        lo_ref[...] = head(po_ref[...], 262)


def _pool_heads(outc2, outo2, hc, ho, dinv, batch2d, cb, ob, headpack):
    return pl.pallas_call(
        _k_pool,
        grid=(GRID,),
        in_specs=[
            pl.BlockSpec((2, BN, H), lambda i: (0, i, 0)),
            pl.BlockSpec((2, BN, H), lambda i: (0, i, 0)),
            pl.BlockSpec((BN, H), _rows),
            pl.BlockSpec((BN, H), _rows),
            pl.BlockSpec((BN, 2), _rows),
            pl.BlockSpec((1, 1, BN), lambda i: (i, 0, 0)),
            pl.BlockSpec((1, H), _full),
            pl.BlockSpec((1, H), _full),
            pl.BlockSpec((524, H), _full),
        ],
        out_specs=[
            pl.BlockSpec((G, NC), _full),
            pl.BlockSpec((G, NC), _full),
            pl.BlockSpec((G, H), _full),
        ],
        out_shape=[
            jax.ShapeDtypeStruct((G, NC), jnp.float32),
            jax.ShapeDtypeStruct((G, NC), jnp.float32),
            jax.ShapeDtypeStruct((G, H), jnp.float32),
        ],
        scratch_shapes=[pltpu.VMEM((G, H), jnp.float32)],
    )(outc2, outo2, hc, ho, dinv, batch2d, cb.reshape(1, H), ob.reshape(1, H),
      headpack)


SC_C = 2
SC_T = 16
NW = SC_C * SC_T
EW = E // NW
CH = 256
TCHUNK = E // CH
NCHUNK = TCHUNK // NW
CREM = TCHUNK - NCHUNK * NW
NPAD = 10240
ZROWS = NPAD // SC_T
ZC = 128
ZN = ZROWS // ZC
OC = 80

_PE = tuple([e for e in range(4) for _ in range(4)])
_HM = tuple(list(range(4)) * 4)


def _sc_gat_body(hw_hbm, aa_hbm, src_hbm, dst_hbm, out_hbm, den_hbm,
                 src_v, dst_v, sidi_v, di_v, ex_v, aag_v,
                 rows_v, out_sh, den_sh, sem):
    c = lax.axis_index("c")
    s = lax.axis_index("s")
    w = s * SC_C + c
    iota = lax.iota(jnp.int32, 16)
    pe = iota >> 2
    hm = iota & 3
    zero = jnp.full((16,), 0.0, jnp.float32)

    def _zr(j, _):
        for t in range(H // 16):
            rows_v[j, pl.ds(16 * t, 16)] = zero
        return 0
    lax.fori_loop(0, CH, _zr, 0)

    def _ze(q, _):
        ex_v[pl.ds(16 * q, 16)] = zero
        return 0
    lax.fori_loop(0, CH * K // 16, _ze, 0)

    for i in range(ZN):
        start = s * ZROWS + i * ZC
        pltpu.sync_copy(rows_v.at[pl.ds(0, ZC)], out_sh.at[pl.ds(start, ZC)])
        pltpu.sync_copy(ex_v.at[pl.ds(0, K * ZC)],
                        den_sh.at[pl.ds(K * start, K * ZC)])
    plsc.subcore_barrier()

    def _chunk(g, _):
        base = (w + NW * g) * CH
        pltpu.sync_copy(src_hbm.at[pl.ds(base, CH)], src_v)
        pltpu.sync_copy(dst_hbm.at[pl.ds(base, CH)], dst_v)
        hrows = pltpu.async_copy(hw_hbm.at[src_v], rows_v, sem)

        def _idx4(q, _):
            idx_e = pe + 4 * q
            srcs = plsc.load_gather(src_v, [idx_e])
            dsts = plsc.load_gather(dst_v, [idx_e])
            sidi_v[pl.ds(16 * q, 16)] = srcs * 8 + hm
            sidi_v[pl.ds(CH * K + 16 * q, 16)] = dsts * 8 + (hm + 4)
            di_v[pl.ds(16 * q, 16)] = dsts * K + hm
            return 0
        lax.fori_loop(0, CH // 4, _idx4, 0)

        pltpu.sync_copy(aa_hbm.at[sidi_v], aag_v)

        def _ex16(q, _):
            t = aag_v[pl.ds(16 * q, 16)] + aag_v[pl.ds(CH * K + 16 * q, 16)]
            alpha = jnp.maximum(t, 0.0) + 0.2 * jnp.minimum(t, 0.0)
            ex_v[pl.ds(16 * q, 16)] = jnp.exp(alpha)
            return 0
        lax.fori_loop(0, CH * K // 16, _ex16, 0)

        hrows.wait()

        def _scale4(q, _):
            exv = ex_v[pl.ds(16 * q, 16)]
            for j in range(4):
                e = 4 * q + j
                for k in range(K):
                    sel = jnp.full((16,), 4 * j + k, jnp.int32)
                    sc = exv.at[sel].get(mode="promise_in_bounds")
                    for t2 in range(2):
                        col = k * DH + 16 * t2
                        rows_v[e, pl.ds(col, 16)] = rows_v[e, pl.ds(col, 16)] * sc
            return 0
        lax.fori_loop(0, CH // 4, _scale4, 0)

        pltpu.sync_copy(ex_v, den_sh.at[di_v], add=True)
        pltpu.sync_copy(rows_v, out_sh.at[dst_v], add=True)
        return 0

    lax.fori_loop(0, NCHUNK, _chunk, 0)

    @pl.when(w < CREM)
    def _extra():
        _chunk(NCHUNK, 0)
    plsc.subcore_barrier()

    for i in range(ZROWS // OC):
        start = s * ZROWS + i * OC

        @pl.when(start < N)
        def _copy_out():
            pltpu.sync_copy(out_sh.at[pl.ds(start, OC)],
                            rows_v.at[pl.ds(0, OC)])
            pltpu.sync_copy(rows_v.at[pl.ds(0, OC)],
                            out_hbm.at[c, pl.ds(start, OC)])
            pltpu.sync_copy(den_sh.at[pl.ds(K * start, K * OC)],
                            ex_v.at[pl.ds(0, K * OC)])
            pltpu.sync_copy(ex_v.at[pl.ds(0, K * OC)],
                            den_hbm.at[pl.ds(c * (N * K) + K * start, K * OC)])


def _edge_gat_sc(hw, aa, src, dst):
    mesh = plsc.VectorSubcoreMesh(core_axis_name="c", subcore_axis_name="s")
    f = pl.kernel(
        _sc_gat_body,
        mesh=mesh,
        compiler_params=pltpu.CompilerParams(needs_layout_passes=False),
        out_type=[
            jax.ShapeDtypeStruct((SC_C, N, H), jnp.float32),
            jax.ShapeDtypeStruct((SC_C * N * K,), jnp.float32),
        ],
        scratch_types=[
            pltpu.VMEM((CH,), jnp.int32),
            pltpu.VMEM((CH,), jnp.int32),
            pltpu.VMEM((CH * 2 * K,), jnp.int32),
            pltpu.VMEM((CH * K,), jnp.int32),
            pltpu.VMEM((CH * K,), jnp.float32),
            pltpu.VMEM((CH * 2 * K,), jnp.float32),
            pltpu.VMEM((CH, H), jnp.float32),
            pltpu.VMEM_SHARED((NPAD, H), jnp.float32),
            pltpu.VMEM_SHARED((NPAD * K,), jnp.float32),
            pltpu.SemaphoreType.DMA,
        ],
    )
    out2, denf = f(hw, aa.reshape(N * 8), src, dst)
    return out2, denf.reshape(SC_C, N, K)


def _sc_att_body(suv_hbm, src_hbm, dst_hbm, ew_hbm, deg_hbm,
                 src_v, dst_v, si_v, dvi_v, di_v, sug_v, svg_v, ew_v,
                 deg_sh, sem):
    c = lax.axis_index("c")
    s = lax.axis_index("s")
    w = s * SC_C + c
    iota = lax.iota(jnp.int32, 16)
    pe2 = iota >> 1
    m2 = iota & 1
    zero = jnp.full((16,), 0.0, jnp.float32)

    def _zw(q, _):
        ew_v[pl.ds(16 * q, 16)] = zero
        return 0
    lax.fori_loop(0, CH * 2 // 16, _zw, 0)
    for i in range(ZN):
        start = s * ZROWS + i * ZC
        pltpu.sync_copy(ew_v.at[pl.ds(0, 2 * ZC)],
                        deg_sh.at[pl.ds(2 * start, 2 * ZC)])
    plsc.subcore_barrier()

    def _chunk(g, _):
        base = (w + NW * g) * CH
        pltpu.sync_copy(src_hbm.at[pl.ds(base, CH)], src_v)
        pltpu.sync_copy(dst_hbm.at[pl.ds(base, CH)], dst_v)

        def _idx8(q, _):
            idx_e = pe2 + 8 * q
            srcs = plsc.load_gather(src_v, [idx_e])
            dsts = plsc.load_gather(dst_v, [idx_e])
            si_v[pl.ds(16 * q, 16)] = srcs * 8
            dvi_v[pl.ds(16 * q, 16)] = dsts * 8 + 1
            di_v[pl.ds(16 * q, 16)] = srcs * 2 + m2
            return 0
        lax.fori_loop(0, CH * 2 // 16, _idx8, 0)

        pltpu.sync_copy(suv_hbm.at[si_v], sug_v)
        pltpu.sync_copy(suv_hbm.at[dvi_v], svg_v)

        def _cw(q, _):
            t = sug_v[pl.ds(16 * q, 16)] + svg_v[pl.ds(16 * q, 16)]
            ewc = 1.0 / (1.0 + jnp.exp(-t))
            ew_v[pl.ds(16 * q, 16)] = jnp.where(m2 == 0, ewc, 1.0 - ewc)
            return 0
        lax.fori_loop(0, CH * 2 // 16, _cw, 0)

        pltpu.sync_copy(ew_v, ew_hbm.at[pl.ds(2 * base, 2 * CH)])
        pltpu.sync_copy(ew_v, deg_sh.at[di_v], add=True)
        return 0

    lax.fori_loop(0, NCHUNK, _chunk, 0)

    @pl.when(w < CREM)
    def _extra():
        _chunk(NCHUNK, 0)
    plsc.subcore_barrier()
    for i in range(ZROWS // OC):
        start = s * ZROWS + i * OC

        @pl.when(start < N)
        def _copy_out():
            pltpu.sync_copy(deg_sh.at[pl.ds(2 * start, 2 * OC)],
                            ew_v.at[pl.ds(0, 2 * OC)])
            pltpu.sync_copy(ew_v.at[pl.ds(0, 2 * OC)],
                            deg_hbm.at[pl.ds(c * (N * 2) + 2 * start, 2 * OC)])


def _edge_att_deg_sc(suv, src, dst):
    mesh = plsc.VectorSubcoreMesh(core_axis_name="c", subcore_axis_name="s")
    f = pl.kernel(
        _sc_att_body,
        mesh=mesh,
        compiler_params=pltpu.CompilerParams(needs_layout_passes=False),
        out_type=[
            jax.ShapeDtypeStruct((E * 2,), jnp.float32),
            jax.ShapeDtypeStruct((SC_C * N * 2,), jnp.float32),
        ],
        scratch_types=[
            pltpu.VMEM((CH,), jnp.int32),
            pltpu.VMEM((CH,), jnp.int32),
            pltpu.VMEM((CH * 2,), jnp.int32),
            pltpu.VMEM((CH * 2,), jnp.int32),
            pltpu.VMEM((CH * 2,), jnp.int32),
            pltpu.VMEM((CH * 2,), jnp.float32),
            pltpu.VMEM((CH * 2,), jnp.float32),
            pltpu.VMEM((CH * 2,), jnp.float32),
            pltpu.VMEM_SHARED((NPAD * 2,), jnp.float32),
            pltpu.SemaphoreType.DMA,
        ],
    )
    ew, degf = f(suv.reshape(N * 8), src, dst)
    return ew, degf.reshape(SC_C, N, 2)


def _make_gcn_body(off):
    def body(h_hbm, dinv_hbm, ew_hbm, src_hbm, dst_hbm, out_hbm,
             src_v, dst_v, sdi_v, ewi_v, dg_v, ewv_v, nm_v,
             rows_v, out_sh, sem):
        c = lax.axis_index("c")
        s = lax.axis_index("s")
        w = s * SC_C + c
        iota = lax.iota(jnp.int32, 16)
        zero = jnp.full((16,), 0.0, jnp.float32)

        def _zr(j, _):
            for t in range(H // 16):
                rows_v[j, pl.ds(16 * t, 16)] = zero
            return 0
        lax.fori_loop(0, CH, _zr, 0)
        for i in range(ZN):
            start = s * ZROWS + i * ZC
            pltpu.sync_copy(rows_v.at[pl.ds(0, ZC)],
                            out_sh.at[pl.ds(start, ZC)])
        plsc.subcore_barrier()

        def _chunk(g, _):
            base = (w + NW * g) * CH
            pltpu.sync_copy(src_hbm.at[pl.ds(base, CH)], src_v)
            pltpu.sync_copy(dst_hbm.at[pl.ds(base, CH)], dst_v)
            hrows = pltpu.async_copy(h_hbm.at[src_v], rows_v, sem)

            def _idx16(q, _):
                idx_e = iota + 16 * q
                srcs = plsc.load_gather(src_v, [idx_e])
                dsts = plsc.load_gather(dst_v, [idx_e])
                sdi_v[pl.ds(16 * q, 16)] = srcs * 2 + off
                sdi_v[pl.ds(CH + 16 * q, 16)] = dsts * 2 + off
                ewi_v[pl.ds(16 * q, 16)] = (base + idx_e) * 2 + off
                return 0
            lax.fori_loop(0, CH // 16, _idx16, 0)

            pltpu.sync_copy(dinv_hbm.at[sdi_v], dg_v)
            pltpu.sync_copy(ew_hbm.at[ewi_v], ewv_v)

            def _nm(q, _):
                sl = pl.ds(16 * q, 16)
                nm_v[sl] = dg_v[sl] * ewv_v[sl] * dg_v[pl.ds(CH + 16 * q, 16)]
                return 0
            lax.fori_loop(0, CH // 16, _nm, 0)

            hrows.wait()

            def _scale(q, _):
                nv = nm_v[pl.ds(16 * q, 16)]
                for j in range(16):
                    e = 16 * q + j
                    sel = jnp.full((16,), j, jnp.int32)
                    sc = nv.at[sel].get(mode="promise_in_bounds")
                    for t2 in range(H // 16):
                        col = 16 * t2
                        rows_v[e, pl.ds(col, 16)] = rows_v[e, pl.ds(col, 16)] * sc
                return 0
            lax.fori_loop(0, CH // 16, _scale, 0)

            pltpu.sync_copy(rows_v, out_sh.at[dst_v], add=True)
            return 0

        lax.fori_loop(0, NCHUNK, _chunk, 0)

        @pl.when(w < CREM)
        def _extra():
            _chunk(NCHUNK, 0)
        plsc.subcore_barrier()
        for i in range(ZROWS // OC):
            start = s * ZROWS + i * OC

            @pl.when(start < N)
            def _copy_out():
                pltpu.sync_copy(out_sh.at[pl.ds(start, OC)],
                                rows_v.at[pl.ds(0, OC)])
                pltpu.sync_copy(rows_v.at[pl.ds(0, OC)],
                                out_hbm.at[c, pl.ds(start, OC)])
    return body


def _edge_gcn_sc(h, dinv, ew, src, dst, off):
    mesh = plsc.VectorSubcoreMesh(core_axis_name="c", subcore_axis_name="s")
    f = pl.kernel(
        _make_gcn_body(off),
        mesh=mesh,
        compiler_params=pltpu.CompilerParams(needs_layout_passes=False),
        out_type=jax.ShapeDtypeStruct((SC_C, N, H), jnp.float32),
        scratch_types=[
            pltpu.VMEM((CH,), jnp.int32),
            pltpu.VMEM((CH,), jnp.int32),
            pltpu.VMEM((CH * 2,), jnp.int32),
            pltpu.VMEM((CH,), jnp.int32),
            pltpu.VMEM((CH * 2,), jnp.float32),
            pltpu.VMEM((CH,), jnp.float32),
            pltpu.VMEM((CH,), jnp.float32),
            pltpu.VMEM((CH, H), jnp.float32),
            pltpu.VMEM_SHARED((NPAD, H), jnp.float32),
            pltpu.SemaphoreType.DMA,
        ],
    )
    return f(h, dinv.reshape(N * 2), ew, src, dst)


def _edge_gat(hw, aa, src, dst):
    t = aa[src, 0:4] + aa[dst, 4:8]
    alpha = jnp.maximum(t, 0) + 0.2 * jnp.minimum(t, 0)
    ex = jnp.exp(alpha)
    den = jnp.zeros((N, K), jnp.float32).at[dst].add(ex)
    hw4 = hw.reshape(N, K, DH)
    out = jnp.zeros((N, K, DH), jnp.float32).at[dst].add(ex[:, :, None] * hw4[src])
    out2 = jnp.stack([out.reshape(N, H), jnp.zeros((N, H), jnp.float32)])
    den2 = jnp.stack([den, jnp.zeros((N, K), jnp.float32)])
    return out2, den2


def _edge_att_deg(suv, src, dst):
    t = suv[src, 0] + suv[dst, 1]
    ewc = jax.nn.sigmoid(t)
    ewo = 1.0 - ewc
    ew = jnp.stack([ewc, ewo], axis=1)
    deg = jnp.zeros((N, 2), jnp.float32).at[src].add(ew)
    deg2 = jnp.stack([deg, jnp.zeros((N, 2), jnp.float32)])
    return ew, deg2


def _edge_gcn(hc, ho, dinv, ew, src, dst):
    nc = dinv[src, 0] * ew[:, 0] * dinv[dst, 0]
    no = dinv[src, 1] * ew[:, 1] * dinv[dst, 1]
    outc = jnp.zeros((N, H), jnp.float32).at[dst].add(nc[:, None] * hc[src])
    outo = jnp.zeros((N, H), jnp.float32).at[dst].add(no[:, None] * ho[src])
    z = jnp.zeros((N, H), jnp.float32)
    return jnp.stack([outc, z]), jnp.stack([outo, z])


def R_layer_norm(x, g, b, eps=1e-5):
    m = x.mean(-1, keepdims=True)
    v = ((x - m) ** 2).mean(-1, keepdims=True)
    return (x - m) * jax.lax.rsqrt(v + eps) * g + b


def kernel(x, params, edge_index, batch, keypoints):
    p = params
    src, dst = edge_index[0], edge_index[1]
    src_i = src.astype(jnp.int32)
    dst_i = dst.astype(jnp.int32)

    def fold(Wn, gn, bn):
        Wp = p[gn][:, None] * p[Wn]
        cp = p[bn] @ p[Wn]
        return Wp, cp

    h = _stage0(x, p['bn_feat_g'], p['bn_feat_b'], p['conv_feat_W'],
                p['conv_feat_b'])

    for i in range(3):
        Wp = p['bn%d_g' % i][:, None] * p['gat%d_W' % i]
        cp = p['bn%d_b' % i] @ p['gat%d_W' % i]
        att_s = p['gat%d_as' % i]
        att_d = p['gat%d_ad' % i]
        Aall = jnp.zeros((H, 8), jnp.float32)
        for k in range(K):
            Aall = Aall.at[DH * k:DH * (k + 1), k].set(att_s[k])
            Aall = Aall.at[DH * k:DH * (k + 1), 4 + k].set(att_d[k])
        hw, aa = _gat_pre(h, Wp, cp, Aall)
        out2, den2 = _edge_gat_sc(hw, aa, src_i, dst_i)
        h = _gat_post(out2, den2, aa, hw, p['gat%d_b' % i])

    eb = p['edge_mlp_b']
    We = jnp.zeros((H, 8), jnp.float32)
    We = We.at[:, 0:2].set(p['edge_mlp_W'][:H])
    We = We.at[:, 2:4].set(p['edge_mlp_W'][H:])
    ebp = jnp.zeros((1, 8), jnp.float32).at[0, 0:2].set(eb)
    Wn = jnp.zeros((H, 8), jnp.float32).at[:, 0:2].set(p['node_mlp_W'])
    nbp = jnp.zeros((1, 8), jnp.float32).at[0, 0:2].set(p['node_mlp_b'])
    Wc, cc = fold('ctx_W', 'bnc_g', 'bnc_b')
    Wo, co = fold('obj_W', 'bno_g', 'bno_b')
    suv, hc, ho = _final(h, We, ebp, Wn, nbp, Wc, cc, Wo, co)

    ew, deg2 = _edge_att_deg_sc(suv, src_i, dst_i)
    dinv = _dinv(deg2)
    outc2 = _edge_gcn_sc(hc, dinv, ew, src_i, dst_i, 0)
    outo2 = _edge_gcn_sc(ho, dinv, ew, src_i, dst_i, 1)

    def packhead(pref):
        rows = [p['ln1%s_g' % pref].reshape(1, H), p['ln1%s_b' % pref].reshape(1, H),
                p['fc1%s_W' % pref], p['fc1%s_b' % pref].reshape(1, H),
                p['ln2%s_g' % pref].reshape(1, H), p['ln2%s_b' % pref].reshape(1, H),
                jnp.zeros((H, H), jnp.float32).at[:, :NC].set(p['fc2%s_W' % pref]),
                jnp.zeros((1, H), jnp.float32).at[0, :NC].set(p['fc2%s_b' % pref])]
        return jnp.concatenate(rows, axis=0)

    headpack = jnp.concatenate([packhead('c'), packhead('o')], axis=0)
    batch2d = batch.astype(jnp.int32).reshape(GRID, 1, BN)
    lc, lo, po = _pool_heads(outc2, outo2, hc, ho, dinv, batch2d,
                             p['ctx_b'], p['obj_b'], headpack)
    return (lc, lo, po)


def _final_jnp(h, We, ebp, Wn, nbp, Wc, cc, Wo, co):
    uv = h @ We + ebp
    su = uv[:, 0:1] - uv[:, 1:2]
    sv = uv[:, 2:3] - uv[:, 3:4]
    suv = jnp.concatenate([su, sv, jnp.zeros((N, 6), jnp.float32)], axis=1)
    na = jax.nn.softmax(h @ Wn[:, 0:2] + nbp[:, 0:2], axis=-1)
    hc = (na[:, 0:1] * h) @ Wc + cc.reshape(1, H)
    ho = (na[:, 1:2] * h) @ Wo + co.reshape(1, H)
    return suv, hc, ho


def _pool_heads_jnp(outc2, outo2, hc, ho, dinv, batch, cb, ob, p):
    xcn = jax.nn.relu(outc2[0] + outc2[1] + dinv[:, 0:1] ** 2 * hc + cb)
    xon = jax.nn.relu(outo2[0] + outo2[1] + dinv[:, 1:2] ** 2 * ho + ob)
    onehot = (batch[None, :] == jnp.arange(G)[:, None]).astype(jnp.float32)
    pc = onehot @ xcn
    po = onehot @ xon

    def ln(z, g, b):
        m = z.mean(-1, keepdims=True)
        v = ((z - m) ** 2).mean(-1, keepdims=True)
        return (z - m) * jax.lax.rsqrt(v + 1e-5) * g + b

    z = ln(pc, p['ln1c_g'], p['ln1c_b'])
    z = jax.nn.relu(z @ p['fc1c_W'] + p['fc1c_b'])
    z = ln(z, p['ln2c_g'], p['ln2c_b'])
    lc = z @ p['fc2c_W'] + p['fc2c_b']
    w = ln(po, p['ln1o_g'], p['ln1o_b'])
    w = jax.nn.relu(w @ p['fc1o_W'] + p['fc1o_b'])
    w = ln(w, p['ln2o_g'], p['ln2o_b'])
    lo = w @ p['fc2o_W'] + p['fc2o_b']
    return lc, lo, po
